# baseline (speedup 1.0000x reference)
import functools

import jax
import jax.numpy as jnp
from jax import lax
from jax.experimental import pallas as pl
from jax.experimental.pallas import tpu as pltpu
from jax.experimental.pallas import tpu_sc as plsc

NUM_CORES = 2
NUM_SUBCORES = 16
NUM_WORKERS = NUM_CORES * NUM_SUBCORES
LANES = 16

BATCH = 16384
DIM = 64
PAIRS_PER_W = BATCH // NUM_WORKERS
ROWS_PER_W = 2 * PAIRS_PER_W
CHUNK = 128
NCHUNK = ROWS_PER_W // CHUNK


def _make_sc_kernel():
    mesh = plsc.VectorSubcoreMesh(core_axis_name="c", subcore_axis_name="s")

    @functools.partial(
        pl.kernel,
        out_type=jax.ShapeDtypeStruct((BATCH,), jnp.float32),
        mesh=mesh,
        compiler_params=pltpu.CompilerParams(
            needs_layout_passes=False, use_tc_tiling_on_sc=False
        ),
        scratch_types=[
            pltpu.VMEM((NCHUNK, CHUNK), jnp.int32),
            pltpu.VMEM((ROWS_PER_W, DIM), jnp.float32),
            pltpu.VMEM((PAIRS_PER_W,), jnp.float32),
            pltpu.SemaphoreType.DMA,
        ],
    )
    def sc_kernel(idx_hbm, table_hbm, out_hbm, idx_v, rows_v, out_v, sem):
        wid = lax.axis_index("s") * NUM_CORES + lax.axis_index("c")

        pltpu.sync_copy(idx_hbm.at[pl.ds(wid * NCHUNK, NCHUNK)], idx_v)

        descs = []
        for j in range(NCHUNK):
            descs.append(
                pltpu.async_copy(
                    table_hbm.at[idx_v.at[j]],
                    rows_v.at[pl.ds(j * CHUNK, CHUNK)],
                    sem,
                )
            )
        for d in descs:
            d.wait()

        lane = lax.iota(jnp.int32, LANES)

        def group_body(g, carry):
            res = jnp.zeros((LANES,), jnp.float32)
            for p in range(LANES):
                row_x = g * (2 * LANES) + 2 * p
                acc = jnp.zeros((LANES,), jnp.float32)
                for c in range(DIM // LANES):
                    xv = rows_v[row_x, pl.ds(c * LANES, LANES)]
                    yv = rows_v[row_x + 1, pl.ds(c * LANES, LANES)]
                    acc = acc + xv * yv
                res = jnp.where(lane == p, jnp.sum(acc), res)
            out_v[pl.ds(g * LANES, LANES)] = 1.0 / (1.0 + jnp.exp(res))
            return carry

        lax.fori_loop(0, PAIRS_PER_W // LANES, group_body, 0)

        pltpu.sync_copy(out_v, out_hbm.at[pl.ds(wid * PAIRS_PER_W, PAIRS_PER_W)])

    return sc_kernel


_SC_KERNEL = _make_sc_kernel()


def kernel(data, embeddings):
    idx2d = data.reshape(BATCH * 2 // CHUNK, CHUNK)
    return _SC_KERNEL(idx2d, embeddings)
